# baseline (speedup 1.0000x reference)
import functools

import jax
import jax.numpy as jnp
from jax import lax
from jax.experimental import pallas as pl
from jax.experimental.pallas import tpu as pltpu
from jax.experimental.pallas import tpu_sc as plsc

NC = 2
NS = 16
NW = NC * NS
LANES = 16
DP = 128


def _bitrev(x, nbits):
    r = 0
    for _ in range(nbits):
        r = (r << 1) | (x & 1)
        x >>= 1
    return r


_GDN = lax.GatherDimensionNumbers(
    offset_dims=(), collapsed_slice_dims=(0,), start_index_map=(0,))


def _perm_xor(v, lane, h):
    return lax.gather(v, (lane ^ h)[:, None], dimension_numbers=_GDN,
                      slice_sizes=(1,),
                      mode=lax.GatherScatterMode.PROMISE_IN_BOUNDS)


def _butterfly(accs, lane):
    n = len(accs)
    assert n & (n - 1) == 0 and n <= LANES
    nbits = n.bit_length() - 1
    vecs = [accs[_bitrev(j, nbits)] for j in range(n)]
    h = LANES // 2
    while h >= n:
        vecs = [v + _perm_xor(v, lane, h) for v in vecs]
        h //= 2
    while len(vecs) > 1:
        nxt = []
        mask = (lane & h) == 0
        for i in range(0, len(vecs), 2):
            fx = vecs[i] + _perm_xor(vecs[i], lane, h)
            fy = vecs[i + 1] + _perm_xor(vecs[i + 1], lane, h)
            nxt.append(jnp.where(mask, fx, fy))
        vecs = nxt
        h //= 2
    return vecs[0]


def _make_kernel(B, K, BC):
    BPW = B // NW
    NCHUNK = BPW // BC
    R = BC * (K + 2)
    NWIN = DP // LANES

    n_left = 1 + (K - LANES)
    assert 0 < n_left <= 8

    mesh = plsc.VectorSubcoreMesh(core_axis_name="c", subcore_axis_name="s")

    @functools.partial(
        pl.kernel,
        compiler_params=pltpu.CompilerParams(use_tc_tiling_on_sc=False),
        out_type=[
            jax.ShapeDtypeStruct((B * LANES,), jnp.float32),
            jax.ShapeDtypeStruct((B * LANES,), jnp.float32),
        ],
        mesh=mesh,
        scratch_types=[
            pltpu.VMEM((R,), jnp.int32),
            pltpu.VMEM((R, DP), jnp.float32),
            pltpu.VMEM((BC * LANES,), jnp.float32),
            pltpu.VMEM((BC * LANES,), jnp.float32),
            pltpu.SemaphoreType.DMA,
        ],
    )
    def k(c_hbm, p_hbm, n_hbm, wb_hbm, neg16_out, left_out,
          iv, rows, neg_stage, left_stage, sem):
        wid = lax.axis_index("s") * NC + lax.axis_index("c")
        base_b = wid * BPW
        lane = lax.iota(jnp.int32, LANES)
        zero = jnp.zeros((LANES,), jnp.float32)

        def chunk_body(ci, carry):
            b0 = base_b + ci * BC
            pltpu.sync_copy(c_hbm.at[pl.ds(b0, BC)], iv.at[pl.ds(0, BC)])
            pltpu.sync_copy(p_hbm.at[pl.ds(b0, BC)], iv.at[pl.ds(BC, BC)])
            pltpu.sync_copy(n_hbm.at[pl.ds(b0 * K, BC * K)],
                            iv.at[pl.ds(2 * BC, BC * K)])
            pltpu.async_copy(wb_hbm.at[iv], rows, sem).wait()

            def b_body(b, carry2):
                cw = [rows[b, pl.ds(w * LANES, LANES)] for w in range(NWIN)]

                def dot_acc(r):
                    acc = cw[0] * rows[r, pl.ds(0, LANES)]
                    for w in range(1, NWIN):
                        acc = acc + cw[w] * rows[r, pl.ds(w * LANES, LANES)]
                    return acc

                accs = [dot_acc(2 * BC + b * K + kk) for kk in range(LANES)]
                neg_stage[pl.ds(b * LANES, LANES)] = _butterfly(accs, lane)

                left = [dot_acc(BC + b)]
                for kk in range(LANES, K):
                    left.append(dot_acc(2 * BC + b * K + kk))
                while len(left) < 8:
                    left.append(zero)
                left_stage[pl.ds(b * LANES, LANES)] = _butterfly(left, lane)
                return carry2

            lax.fori_loop(0, BC, b_body, 0)
            pltpu.sync_copy(neg_stage,
                            neg16_out.at[pl.ds(b0 * LANES, BC * LANES)])
            pltpu.sync_copy(left_stage,
                            left_out.at[pl.ds(b0 * LANES, BC * LANES)])
            return carry

        lax.fori_loop(0, NCHUNK, chunk_body, 0)

    return k


def _pad_both_tc(wc, wx, dp):
    V, D = wc.shape
    BS = 8000
    assert V % BS == 0

    def body(c_ref, x_ref, o_ref):
        t = pl.program_id(0)
        z = jnp.zeros((BS, dp - D), jnp.float32)

        @pl.when(t == 0)
        def _():
            o_ref[0, :, pl.ds(0, D)] = c_ref[...]
            o_ref[0, :, pl.ds(D, dp - D)] = z

        @pl.when(t == 1)
        def _():
            o_ref[0, :, pl.ds(0, D)] = x_ref[...]
            o_ref[0, :, pl.ds(D, dp - D)] = z

    out = pl.pallas_call(
        body,
        grid=(2, V // BS),
        in_specs=[
            pl.BlockSpec((BS, D), lambda t, i: (jnp.where(t == 0, i, 0), 0)),
            pl.BlockSpec((BS, D), lambda t, i: (jnp.where(t == 0, 0, i), 0)),
        ],
        out_specs=pl.BlockSpec((1, BS, dp), lambda t, i: (t, i, 0)),
        out_shape=jax.ShapeDtypeStruct((2, V, dp), jnp.float32),
    )(wc, wx)
    return out.reshape(2 * V, dp)


def kernel(center, pos_context, neg_contexts, W_center, W_context):
    B = center.shape[0]
    K = neg_contexts.shape[1]
    D = W_center.shape[1]
    V = W_center.shape[0]
    c = center.astype(jnp.int32)
    p = pos_context.astype(jnp.int32) + V
    n = neg_contexts.astype(jnp.int32).reshape(-1) + V
    wb = _pad_both_tc(W_center.astype(jnp.float32),
                      W_context.astype(jnp.float32), DP)

    k = _make_kernel(B, K, BC=32)
    neg16_flat, left_flat = k(c, p, n, wb)
    neg16 = neg16_flat.reshape(B, LANES)
    left = left_flat.reshape(B, LANES)
    pos_dot = left[:, 0:1]
    neg_dot = jnp.concatenate([neg16, left[:, 1:1 + (K - LANES)]], axis=1)
    return pos_dot, neg_dot


# speedup vs baseline: 1.0271x; 1.0271x over previous
import functools

import jax
import jax.numpy as jnp
from jax import lax
from jax.experimental import pallas as pl
from jax.experimental.pallas import tpu as pltpu
from jax.experimental.pallas import tpu_sc as plsc

NC = 2
NS = 16
NW = NC * NS
LANES = 16
DP = 128


def _bitrev(x, nbits):
    r = 0
    for _ in range(nbits):
        r = (r << 1) | (x & 1)
        x >>= 1
    return r


_GDN = lax.GatherDimensionNumbers(
    offset_dims=(), collapsed_slice_dims=(0,), start_index_map=(0,))


def _perm_xor(v, lane, h):
    return lax.gather(v, (lane ^ h)[:, None], dimension_numbers=_GDN,
                      slice_sizes=(1,),
                      mode=lax.GatherScatterMode.PROMISE_IN_BOUNDS)


def _butterfly(accs, lane):
    n = len(accs)
    assert n & (n - 1) == 0 and n <= LANES
    nbits = n.bit_length() - 1
    vecs = [accs[_bitrev(j, nbits)] for j in range(n)]
    h = LANES // 2
    while h >= n:
        vecs = [v + _perm_xor(v, lane, h) for v in vecs]
        h //= 2
    while len(vecs) > 1:
        nxt = []
        mask = (lane & h) == 0
        for i in range(0, len(vecs), 2):
            fx = vecs[i] + _perm_xor(vecs[i], lane, h)
            fy = vecs[i + 1] + _perm_xor(vecs[i + 1], lane, h)
            nxt.append(jnp.where(mask, fx, fy))
        vecs = nxt
        h //= 2
    return vecs[0]


def _make_kernel(B, K, BC):
    BPW = B // NW
    NCHUNK = BPW // BC
    R = BC * (K + 2)
    NWIN = DP // LANES

    n_left = 1 + (K - LANES)
    assert 0 < n_left <= 8

    mesh = plsc.VectorSubcoreMesh(core_axis_name="c", subcore_axis_name="s")

    @functools.partial(
        pl.kernel,
        compiler_params=pltpu.CompilerParams(use_tc_tiling_on_sc=False),
        out_type=[
            jax.ShapeDtypeStruct((B * LANES,), jnp.float32),
            jax.ShapeDtypeStruct((B * LANES,), jnp.float32),
        ],
        mesh=mesh,
        scratch_types=[
            pltpu.VMEM((2, R), jnp.int32),
            pltpu.VMEM((2, R, DP), jnp.float32),
            pltpu.VMEM((BC * LANES,), jnp.float32),
            pltpu.VMEM((BC * LANES,), jnp.float32),
            pltpu.SemaphoreType.DMA,
            pltpu.SemaphoreType.DMA,
        ],
    )
    def k(c_hbm, p_hbm, n_hbm, wb_hbm, neg16_out, left_out,
          iv, rows, neg_stage, left_stage, sem0, sem1):
        wid = lax.axis_index("s") * NC + lax.axis_index("c")
        base_b = wid * BPW
        lane = lax.iota(jnp.int32, LANES)
        zero = jnp.zeros((LANES,), jnp.float32)
        bufs = [(iv.at[0], rows.at[0], sem0), (iv.at[1], rows.at[1], sem1)]

        def stage_and_fire(ci, buf):
            ivb, rowsb, semb = bufs[buf]
            b0 = base_b + ci * BC
            pltpu.sync_copy(c_hbm.at[pl.ds(b0, BC)], ivb.at[pl.ds(0, BC)])
            pltpu.sync_copy(p_hbm.at[pl.ds(b0, BC)], ivb.at[pl.ds(BC, BC)])
            pltpu.sync_copy(n_hbm.at[pl.ds(b0 * K, BC * K)],
                            ivb.at[pl.ds(2 * BC, BC * K)])
            pltpu.async_copy(wb_hbm.at[ivb], rowsb, semb)

        def compute_chunk(ci, buf):
            ivb, rowsb, semb = bufs[buf]
            pltpu.make_async_copy(wb_hbm.at[ivb], rowsb, semb).wait()
            b0 = base_b + ci * BC

            def b_body(b, carry2):
                cw = [rowsb[b, pl.ds(w * LANES, LANES)] for w in range(NWIN)]

                def dot_acc(r):
                    acc = cw[0] * rowsb[r, pl.ds(0, LANES)]
                    for w in range(1, NWIN):
                        acc = acc + cw[w] * rowsb[r, pl.ds(w * LANES, LANES)]
                    return acc

                accs = [dot_acc(2 * BC + b * K + kk) for kk in range(LANES)]
                neg_stage[pl.ds(b * LANES, LANES)] = _butterfly(accs, lane)

                left = [dot_acc(BC + b)]
                for kk in range(LANES, K):
                    left.append(dot_acc(2 * BC + b * K + kk))
                while len(left) < 8:
                    left.append(zero)
                left_stage[pl.ds(b * LANES, LANES)] = _butterfly(left, lane)
                return carry2

            lax.fori_loop(0, BC, b_body, 0)
            pltpu.sync_copy(neg_stage,
                            neg16_out.at[pl.ds(b0 * LANES, BC * LANES)])
            pltpu.sync_copy(left_stage,
                            left_out.at[pl.ds(b0 * LANES, BC * LANES)])

        assert NCHUNK % 2 == 0
        stage_and_fire(0, 0)

        def pair_body(m, carry):
            ca = 2 * m
            stage_and_fire(ca + 1, 1)
            compute_chunk(ca, 0)

            @pl.when(ca + 2 < NCHUNK)
            def _():
                stage_and_fire(ca + 2, 0)

            compute_chunk(ca + 1, 1)
            return carry

        lax.fori_loop(0, NCHUNK // 2, pair_body, 0)

    return k


def _pad_both_tc(wc, wx, dp):
    V, D = wc.shape
    BS = 8000
    assert V % BS == 0

    def body(c_ref, x_ref, o_ref):
        t = pl.program_id(0)
        z = jnp.zeros((BS, dp - D), jnp.float32)

        @pl.when(t == 0)
        def _():
            o_ref[0, :, pl.ds(0, D)] = c_ref[...]
            o_ref[0, :, pl.ds(D, dp - D)] = z

        @pl.when(t == 1)
        def _():
            o_ref[0, :, pl.ds(0, D)] = x_ref[...]
            o_ref[0, :, pl.ds(D, dp - D)] = z

    out = pl.pallas_call(
        body,
        grid=(2, V // BS),
        in_specs=[
            pl.BlockSpec((BS, D), lambda t, i: (jnp.where(t == 0, i, 0), 0)),
            pl.BlockSpec((BS, D), lambda t, i: (jnp.where(t == 0, 0, i), 0)),
        ],
        out_specs=pl.BlockSpec((1, BS, dp), lambda t, i: (t, i, 0)),
        out_shape=jax.ShapeDtypeStruct((2, V, dp), jnp.float32),
    )(wc, wx)
    return out.reshape(2 * V, dp)


def kernel(center, pos_context, neg_contexts, W_center, W_context):
    B = center.shape[0]
    K = neg_contexts.shape[1]
    D = W_center.shape[1]
    V = W_center.shape[0]
    c = center.astype(jnp.int32)
    p = pos_context.astype(jnp.int32) + V
    n = neg_contexts.astype(jnp.int32).reshape(-1) + V
    wb = _pad_both_tc(W_center.astype(jnp.float32),
                      W_context.astype(jnp.float32), DP)

    k = _make_kernel(B, K, BC=16)
    neg16_flat, left_flat = k(c, p, n, wb)
    neg16 = neg16_flat.reshape(B, LANES)
    left = left_flat.reshape(B, LANES)
    pos_dot = left[:, 0:1]
    neg_dot = jnp.concatenate([neg16, left[:, 1:1 + (K - LANES)]], axis=1)
    return pos_dot, neg_dot
